# baseline (speedup 1.0000x reference)
import functools

import jax
import jax.numpy as jnp
from jax import lax
from jax.experimental import pallas as pl
from jax.experimental.pallas import tpu as pltpu
from jax.experimental.pallas import tpu_sc as plsc

NUM_FIELDS = 26
EMBED_DIM = 32
BATCH = 16384
ROWS128 = (2600000 * EMBED_DIM) // 128
NC, NS = 2, 16
NW = NC * NS
BW = BATCH // NW
NG = BW // 16
L = 16


def _emb_body(batcht_hbm, offs_hbm, table_hbm, out_hbm,
              idx_c, glob_b, rowidx_b, ov_c, rows_v, out_t, gsem):
    wid = lax.axis_index("s") * NC + lax.axis_index("c")
    b0 = wid * BW
    lanes = lax.iota(jnp.int32, L)

    def field_body(f, carry):
        pltpu.sync_copy(batcht_hbm.at[pl.ds(f, 1), pl.ds(b0, BW)], idx_c)
        pltpu.sync_copy(offs_hbm.at[pl.ds(f, 1)], ov_c)
        ov = ov_c[0, :]

        def prep_body(g, c):
            gl = idx_c[0, pl.ds(g * L, L)] + ov
            glob_b[pl.ds(g * L, L)] = gl
            rowidx_b[pl.ds(g * L, L)] = lax.shift_right_logical(gl, 2)
            return c

        lax.fori_loop(0, NG, prep_body, 0)

        copies = []
        for s in range(4):
            src = table_hbm.at[rowidx_b.at[pl.ds(s * 128, 128)]]
            copies.append(
                pltpu.async_copy(src, rows_v.at[pl.ds(s * 128, 128), :], gsem))
        for cp in copies:
            cp.wait()

        def ext_body(g, c):
            row_sel = lanes + g * L
            gl = glob_b[pl.ds(g * L, L)]
            col0 = (gl & 3) * EMBED_DIM
            for t in range(EMBED_DIM):
                vals = plsc.load_gather(rows_v, [row_sel, col0 + t])
                out_t[0, t, pl.ds(g * L, L)] = vals
            return c

        lax.fori_loop(0, NG, ext_body, 0)

        pltpu.sync_copy(out_t, out_hbm.at[pl.ds(f, 1), :, pl.ds(b0, BW)])
        return carry

    lax.fori_loop(0, NUM_FIELDS, field_body, 0)


def kernel(batch, offsets, table):
    batcht = batch.astype(jnp.int32).T
    offs16 = jnp.tile(offsets.astype(jnp.int32)[:, None], (1, L))
    table128 = table.reshape(ROWS128, 128)

    mesh = plsc.VectorSubcoreMesh(core_axis_name="c", subcore_axis_name="s")
    run = functools.partial(
        pl.kernel,
        mesh=mesh,
        out_type=jax.ShapeDtypeStruct((NUM_FIELDS, EMBED_DIM, BATCH),
                                      jnp.float32),
        scratch_types=[
            pltpu.VMEM((1, BW), jnp.int32),
            pltpu.VMEM((BW,), jnp.int32),
            pltpu.VMEM((BW,), jnp.int32),
            pltpu.VMEM((1, L), jnp.int32),
            pltpu.VMEM((BW, 128), jnp.float32),
            pltpu.VMEM((1, EMBED_DIM, BW), jnp.float32),
            pltpu.SemaphoreType.DMA,
        ],
        compiler_params=pltpu.CompilerParams(use_tc_tiling_on_sc=True,
                                             needs_layout_passes=False),
    )(_emb_body)
    out_fdb = run(batcht, offs16, table128)
    return out_fdb.transpose(2, 0, 1)
